# baseline (speedup 1.0000x reference)
import jax
import jax.numpy as jnp
from jax.experimental import pallas as pl

_F32 = jnp.float32
_BF16 = jnp.bfloat16
_BPS = 4


def _chamfer_body(p_ref, ab_ref, out_ref):
    for j in range(_BPS):
        at = p_ref[j, :4, :]
        bt = p_ref[j, 4:, :]
        a16 = ab_ref[j, :16, :]
        b16 = ab_ref[j, 16:, :]
        b2 = jnp.sum(bt * bt, axis=0, keepdims=True)
        p1 = b2.astype(_BF16)
        r1 = b2 - p1.astype(_F32)
        p2 = r1.astype(_BF16)
        p3 = (r1 - p2.astype(_F32)).astype(_BF16)
        row = jax.lax.broadcasted_iota(jnp.int32, b16.shape, 0)
        b16 = jnp.where(row == 9, jnp.broadcast_to(p1, b16.shape), b16)
        b16 = jnp.where(row == 10, jnp.broadcast_to(p2, b16.shape), b16)
        b16 = jnp.where(row == 11, jnp.broadcast_to(p3, b16.shape), b16)
        d = jax.lax.dot_general(
            a16, b16, (((0,), (0,)), ((), ())),
            preferred_element_type=_F32)
        m = jnp.min(d, axis=1)
        a2 = jnp.sum(at * at, axis=0)
        loss = jnp.mean(a2 + m)
        total = loss if j == 0 else total + loss
    out_ref[...] = jnp.broadcast_to(total, out_ref.shape)


def kernel(adv_pc, ori_pc):
    B, K, _ = adv_pc.shape
    pts = jnp.concatenate(
        [adv_pc, jnp.zeros((B, K, 1), _F32),
         ori_pc, jnp.zeros((B, K, 1), _F32)], axis=2)
    p = pts.transpose(0, 2, 1)
    c = -2.0 * p[:, :3, :]
    c_hi = c.astype(_BF16)
    c_lo = (c - c_hi.astype(_F32)).astype(_BF16)
    bq = p[:, 4:7, :]
    b_hi = bq.astype(_BF16)
    b_lo = (bq - b_hi.astype(_F32)).astype(_BF16)
    ab = jnp.concatenate(
        [c_hi, c_lo, c_hi, jnp.ones((B, 3, K), _BF16), jnp.zeros((B, 4, K), _BF16),
         b_hi, b_hi, b_lo, jnp.zeros((B, 7, K), _BF16)], axis=1)
    steps = B // _BPS
    out = pl.pallas_call(
        _chamfer_body,
        grid=(steps,),
        in_specs=[
            pl.BlockSpec((_BPS, 8, K), lambda b: (b, 0, 0)),
            pl.BlockSpec((_BPS, 32, K), lambda b: (b, 0, 0)),
        ],
        out_specs=pl.BlockSpec((1, 1, 128), lambda b: (b, 0, 0)),
        out_shape=jax.ShapeDtypeStruct((steps, 1, 128), jnp.float32),
    )(p, ab)
    return jnp.sum(out[:, 0, 0]) / B


# speedup vs baseline: 1.7206x; 1.7206x over previous
import jax
import jax.numpy as jnp
from jax.experimental import pallas as pl

_BPS = 8


def _chamfer_body(p_ref, out_ref):
    for j in range(_BPS):
        at = p_ref[j, :4, :]
        bt = p_ref[j, 4:, :]
        row_a = jax.lax.broadcasted_iota(jnp.int32, at.shape, 0)
        a_aug = jnp.where(row_a == 3, 1.0, -2.0 * at)
        b2 = jnp.sum(bt * bt, axis=0, keepdims=True)
        row_b = jax.lax.broadcasted_iota(jnp.int32, bt.shape, 0)
        bt_aug = jnp.where(row_b == 3, b2, bt)
        d = jax.lax.dot_general(
            a_aug, bt_aug, (((0,), (0,)), ((), ())),
            preferred_element_type=jnp.float32)
        m = jnp.min(d, axis=1)
        a2 = jnp.sum(at * at, axis=0)
        loss = jnp.mean(a2 + m)
        total = loss if j == 0 else total + loss
    out_ref[...] = jnp.broadcast_to(total, out_ref.shape)


def kernel(adv_pc, ori_pc):
    B, K, _ = adv_pc.shape
    pts = jnp.concatenate(
        [adv_pc, jnp.zeros((B, K, 1), jnp.float32),
         ori_pc, jnp.zeros((B, K, 1), jnp.float32)], axis=2)
    p = pts.transpose(0, 2, 1)
    steps = B // _BPS
    out = pl.pallas_call(
        _chamfer_body,
        grid=(steps,),
        in_specs=[pl.BlockSpec((_BPS, 8, K), lambda b: (b, 0, 0))],
        out_specs=pl.BlockSpec((1, 1, 128), lambda b: (b, 0, 0)),
        out_shape=jax.ShapeDtypeStruct((steps, 1, 128), jnp.float32),
    )(p)
    return jnp.sum(out[:, 0, 0]) / B
